# Initial kernel scaffold; baseline (speedup 1.0000x reference)
import functools

import jax
import jax.numpy as jnp
from jax import lax
from jax.experimental import pallas as pl
from jax.experimental.pallas import tpu as pltpu, tpu_sc as plsc

N = 100000
G = 5000
NUM_SUBCORES = 16
SLAB = 6272
N_PAD = SLAB * NUM_SUBCORES
HALF = SLAB // 2
G_ROWS = 40
CHUNKS_ACC = SLAB // 16
CHUNKS_OUT = HALF // 16


def _body(e_hbm, s_hbm, ids_hbm, zeros_hbm, iota_hbm, out_hbm,
          ids_v, e_v, s_v, sinv_v, acc_s, acc_e, sum_s, sum_e, iota_v, outq_v,
          sh_s, sh_e):
    cid = lax.axis_index("c")
    sid = lax.axis_index("s")
    base = sid * SLAB

    pltpu.sync_copy(e_hbm.at[pl.ds(base, SLAB)], e_v)
    pltpu.sync_copy(s_hbm.at[pl.ds(base, SLAB)], s_v)
    pltpu.sync_copy(ids_hbm.at[pl.ds(base, SLAB)], ids_v)
    pltpu.sync_copy(zeros_hbm, acc_s)
    pltpu.sync_copy(zeros_hbm, acc_e)
    pltpu.sync_copy(iota_hbm, iota_v)

    @pl.when(sid == 0)
    def _():
        pltpu.sync_copy(zeros_hbm, sh_s)
        pltpu.sync_copy(zeros_hbm, sh_e)

    plsc.subcore_barrier()

    def acc_body(i, carry):
        off = i * 16
        ids16 = ids_v[pl.ds(off, 16)]
        e16 = e_v[pl.ds(off, 16)]
        s16 = s_v[pl.ds(off, 16)]
        sinv = 1.0 / s16
        es = e16 * sinv
        sinv_v[pl.ds(off, 16)] = sinv
        row = lax.shift_right_logical(ids16, 7)
        col = jnp.bitwise_and(ids16, 127)
        plsc.addupdate_scatter(acc_s, [row, col], sinv)
        plsc.addupdate_scatter(acc_e, [row, col], es)
        return carry

    lax.fori_loop(0, CHUNKS_ACC, acc_body, 0)

    pltpu.sync_copy(acc_s, sh_s.at[iota_v], add=True)
    pltpu.sync_copy(acc_e, sh_e.at[iota_v], add=True)

    plsc.subcore_barrier()

    pltpu.sync_copy(sh_s, sum_s)
    pltpu.sync_copy(sh_e, sum_e)

    obase = cid * HALF

    def out_body(i, carry):
        off = obase + i * 16
        ids16 = ids_v[pl.ds(off, 16)]
        row = lax.shift_right_logical(ids16, 7)
        col = jnp.bitwise_and(ids16, 127)
        g_s = plsc.load_gather(sum_s, [row, col])
        g_e = plsc.load_gather(sum_e, [row, col])
        sinv = sinv_v[pl.ds(off, 16)]
        e16 = e_v[pl.ds(off, 16)]
        q = sinv * (g_e / g_s) - e16 * sinv
        outq_v[pl.ds(i * 16, 16)] = q
        return carry

    lax.fori_loop(0, CHUNKS_OUT, out_body, 0)

    pltpu.sync_copy(outq_v, out_hbm.at[pl.ds(base + obase, HALF)])


@functools.partial(
    pl.kernel,
    out_type=jax.ShapeDtypeStruct((N_PAD,), jnp.float32),
    mesh=plsc.VectorSubcoreMesh(core_axis_name="c", subcore_axis_name="s"),
    scratch_types=[
        pltpu.VMEM((SLAB,), jnp.int32),
        pltpu.VMEM((SLAB,), jnp.float32),
        pltpu.VMEM((SLAB,), jnp.float32),
        pltpu.VMEM((SLAB,), jnp.float32),
        pltpu.VMEM((G_ROWS, 128), jnp.float32),
        pltpu.VMEM((G_ROWS, 128), jnp.float32),
        pltpu.VMEM((G_ROWS, 128), jnp.float32),
        pltpu.VMEM((G_ROWS, 128), jnp.float32),
        pltpu.VMEM((G_ROWS,), jnp.int32),
        pltpu.VMEM((HALF,), jnp.float32),
        pltpu.VMEM_SHARED((G_ROWS, 128), jnp.float32),
        pltpu.VMEM_SHARED((G_ROWS, 128), jnp.float32),
    ],
)
def _sc_kernel(e_hbm, s_hbm, ids_hbm, zeros_hbm, iota_hbm, out_hbm, *scratch):
    _body(e_hbm, s_hbm, ids_hbm, zeros_hbm, iota_hbm, out_hbm, *scratch)


def kernel(e, s, segment_ids):
    pad = N_PAD - N
    e1 = jnp.concatenate([e.reshape(-1), jnp.zeros((pad,), jnp.float32)])
    s1 = jnp.concatenate([s.reshape(-1), jnp.ones((pad,), jnp.float32)])
    ids1 = jnp.concatenate(
        [segment_ids, jnp.full((pad,), G, jnp.int32)])
    zeros = jnp.zeros((G_ROWS, 128), jnp.float32)
    iota = jnp.arange(G_ROWS, dtype=jnp.int32)
    q = _sc_kernel(e1, s1, ids1, zeros, iota)
    return q[:N].reshape(N, 1)


# speedup vs baseline: 32.9321x; 32.9321x over previous
import functools

import jax
import jax.numpy as jnp
from jax import lax
from jax.experimental import pallas as pl
from jax.experimental.pallas import tpu as pltpu, tpu_sc as plsc

N = 100000
G = 5000
NUM_SUBCORES = 16
SLAB = 6272
N_PAD = SLAB * NUM_SUBCORES
HALF = SLAB // 2
G_ROWS = 40
CHUNKS_ACC = SLAB // 16
CHUNKS_OUT = HALF // 16


def _body(e_hbm, s_hbm, ids_hbm, zeros_hbm, iota_hbm, out_hbm,
          ids_v, e_v, s_v, sinv_v, acc_s, acc_e, sum_s, sum_e, iota_v, outq_v,
          sh_s, sh_e):
    cid = lax.axis_index("c")
    sid = lax.axis_index("s")
    base = sid * SLAB

    pltpu.sync_copy(e_hbm.at[pl.ds(base, SLAB)], e_v)
    pltpu.sync_copy(s_hbm.at[pl.ds(base, SLAB)], s_v)
    pltpu.sync_copy(ids_hbm.at[pl.ds(base, SLAB)], ids_v)
    pltpu.sync_copy(zeros_hbm, acc_s)
    pltpu.sync_copy(zeros_hbm, acc_e)
    pltpu.sync_copy(iota_hbm, iota_v)

    @pl.when(sid == 0)
    def _():
        pltpu.sync_copy(zeros_hbm, sh_s)
        pltpu.sync_copy(zeros_hbm, sh_e)

    plsc.subcore_barrier()

    def acc_body(i, carry):
        off = i * 16
        ids16 = ids_v[pl.ds(off, 16)]
        e16 = e_v[pl.ds(off, 16)]
        s16 = s_v[pl.ds(off, 16)]
        sinv = 1.0 / s16
        es = e16 * sinv
        sinv_v[pl.ds(off, 16)] = sinv
        row = lax.shift_right_logical(ids16, 7)
        col = jnp.bitwise_and(ids16, 127)
        plsc.addupdate_scatter(acc_s, [row, col], sinv)
        plsc.addupdate_scatter(acc_e, [row, col], es)
        return carry

    lax.fori_loop(0, CHUNKS_ACC, acc_body, 0)

    pltpu.sync_copy(acc_s, sh_s.at[iota_v], add=True)
    pltpu.sync_copy(acc_e, sh_e.at[iota_v], add=True)

    plsc.subcore_barrier()

    pltpu.sync_copy(sh_s, sum_s)
    pltpu.sync_copy(sh_e, sum_e)

    obase = cid * HALF

    def out_body(i, carry):
        off = obase + i * 16
        ids16 = ids_v[pl.ds(off, 16)]
        row = lax.shift_right_logical(ids16, 7)
        col = jnp.bitwise_and(ids16, 127)
        g_s = plsc.load_gather(sum_s, [row, col])
        g_e = plsc.load_gather(sum_e, [row, col])
        sinv = sinv_v[pl.ds(off, 16)]
        e16 = e_v[pl.ds(off, 16)]
        q = sinv * (g_e / g_s) - e16 * sinv
        outq_v[pl.ds(i * 16, 16)] = q
        return carry

    lax.fori_loop(0, CHUNKS_OUT, out_body, 0)

    pltpu.sync_copy(outq_v, out_hbm.at[pl.ds(base + obase, HALF)])


@functools.partial(
    pl.kernel,
    out_type=jax.ShapeDtypeStruct((N_PAD,), jnp.float32),
    mesh=plsc.VectorSubcoreMesh(core_axis_name="c", subcore_axis_name="s"),
    compiler_params=pltpu.CompilerParams(needs_layout_passes=False),
    scratch_types=[
        pltpu.VMEM((SLAB,), jnp.int32),
        pltpu.VMEM((SLAB,), jnp.float32),
        pltpu.VMEM((SLAB,), jnp.float32),
        pltpu.VMEM((SLAB,), jnp.float32),
        pltpu.VMEM((G_ROWS, 128), jnp.float32),
        pltpu.VMEM((G_ROWS, 128), jnp.float32),
        pltpu.VMEM((G_ROWS, 128), jnp.float32),
        pltpu.VMEM((G_ROWS, 128), jnp.float32),
        pltpu.VMEM((G_ROWS,), jnp.int32),
        pltpu.VMEM((HALF,), jnp.float32),
        pltpu.VMEM_SHARED((G_ROWS, 128), jnp.float32),
        pltpu.VMEM_SHARED((G_ROWS, 128), jnp.float32),
    ],
)
def _sc_kernel(e_hbm, s_hbm, ids_hbm, zeros_hbm, iota_hbm, out_hbm, *scratch):
    _body(e_hbm, s_hbm, ids_hbm, zeros_hbm, iota_hbm, out_hbm, *scratch)


def kernel(e, s, segment_ids):
    pad = N_PAD - N
    e1 = jnp.concatenate([e.reshape(-1), jnp.zeros((pad,), jnp.float32)])
    s1 = jnp.concatenate([s.reshape(-1), jnp.ones((pad,), jnp.float32)])
    ids1 = jnp.concatenate(
        [segment_ids, jnp.full((pad,), G, jnp.int32)])
    zeros = jnp.zeros((G_ROWS, 128), jnp.float32)
    iota = jnp.arange(G_ROWS, dtype=jnp.int32)
    q = _sc_kernel(e1, s1, ids1, zeros, iota)
    return q[:N].reshape(N, 1)


# speedup vs baseline: 33.7824x; 1.0258x over previous
import functools

import jax
import jax.numpy as jnp
from jax import lax
from jax.experimental import pallas as pl
from jax.experimental.pallas import tpu as pltpu, tpu_sc as plsc

N = 100000
G = 5000
NUM_SUBCORES = 16
SLAB = 6272
N_PAD = SLAB * NUM_SUBCORES
HALF = SLAB // 2
G_ROWS = 40
CHUNKS_ACC = SLAB // 16
CHUNKS_OUT = HALF // 16


def _body(e_hbm, s_hbm, ids_hbm, zeros_hbm, iota_hbm, out_hbm,
          ids_v, e_v, s_v, sinv_v, acc_s, acc_e, sum_s, sum_e, iota_v, outq_v,
          sh_s, sh_e):
    sid = lax.axis_index("s")
    base = sid * SLAB

    pltpu.sync_copy(e_hbm.at[pl.ds(base, SLAB)], e_v)
    pltpu.sync_copy(s_hbm.at[pl.ds(base, SLAB)], s_v)
    pltpu.sync_copy(ids_hbm.at[pl.ds(base, SLAB)], ids_v)
    pltpu.sync_copy(zeros_hbm, acc_s)
    pltpu.sync_copy(zeros_hbm, acc_e)
    pltpu.sync_copy(iota_hbm, iota_v)

    @pl.when(sid == 0)
    def _():
        pltpu.sync_copy(zeros_hbm, sh_s)
        pltpu.sync_copy(zeros_hbm, sh_e)

    plsc.subcore_barrier()

    def acc_body(i, carry):
        off = i * 16
        ids16 = ids_v[pl.ds(off, 16)]
        e16 = e_v[pl.ds(off, 16)]
        s16 = s_v[pl.ds(off, 16)]
        sinv = 1.0 / s16
        es = e16 * sinv
        sinv_v[pl.ds(off, 16)] = sinv
        row = lax.shift_right_logical(ids16, 7)
        col = jnp.bitwise_and(ids16, 127)
        plsc.addupdate_scatter(acc_s, [row, col], sinv)
        plsc.addupdate_scatter(acc_e, [row, col], es)
        return carry

    lax.fori_loop(0, CHUNKS_ACC, acc_body, 0)

    pltpu.sync_copy(acc_s, sh_s.at[iota_v], add=True)
    pltpu.sync_copy(acc_e, sh_e.at[iota_v], add=True)

    plsc.subcore_barrier()

    pltpu.sync_copy(sh_s, sum_s)
    pltpu.sync_copy(sh_e, sum_e)

    def out_body(i, carry):
        off = i * 16
        ids16 = ids_v[pl.ds(off, 16)]
        row = lax.shift_right_logical(ids16, 7)
        col = jnp.bitwise_and(ids16, 127)
        g_s = plsc.load_gather(sum_s, [row, col])
        g_e = plsc.load_gather(sum_e, [row, col])
        sinv = sinv_v[pl.ds(off, 16)]
        e16 = e_v[pl.ds(off, 16)]
        q = sinv * (g_e / g_s) - e16 * sinv
        outq_v[pl.ds(i * 16, 16)] = q
        return carry

    lax.fori_loop(0, CHUNKS_ACC, out_body, 0)

    pltpu.sync_copy(outq_v, out_hbm.at[pl.ds(base, SLAB)])


@functools.partial(
    pl.kernel,
    out_type=jax.ShapeDtypeStruct((N_PAD,), jnp.float32),
    mesh=plsc.VectorSubcoreMesh(core_axis_name="c", subcore_axis_name="s", num_cores=1),
    compiler_params=pltpu.CompilerParams(needs_layout_passes=False),
    scratch_types=[
        pltpu.VMEM((SLAB,), jnp.int32),
        pltpu.VMEM((SLAB,), jnp.float32),
        pltpu.VMEM((SLAB,), jnp.float32),
        pltpu.VMEM((SLAB,), jnp.float32),
        pltpu.VMEM((G_ROWS, 128), jnp.float32),
        pltpu.VMEM((G_ROWS, 128), jnp.float32),
        pltpu.VMEM((G_ROWS, 128), jnp.float32),
        pltpu.VMEM((G_ROWS, 128), jnp.float32),
        pltpu.VMEM((G_ROWS,), jnp.int32),
        pltpu.VMEM((SLAB,), jnp.float32),
        pltpu.VMEM_SHARED((G_ROWS, 128), jnp.float32),
        pltpu.VMEM_SHARED((G_ROWS, 128), jnp.float32),
    ],
)
def _sc_kernel(e_hbm, s_hbm, ids_hbm, zeros_hbm, iota_hbm, out_hbm, *scratch):
    _body(e_hbm, s_hbm, ids_hbm, zeros_hbm, iota_hbm, out_hbm, *scratch)


def kernel(e, s, segment_ids):
    pad = N_PAD - N
    e1 = jnp.concatenate([e.reshape(-1), jnp.zeros((pad,), jnp.float32)])
    s1 = jnp.concatenate([s.reshape(-1), jnp.ones((pad,), jnp.float32)])
    ids1 = jnp.concatenate(
        [segment_ids, jnp.full((pad,), G, jnp.int32)])
    zeros = jnp.zeros((G_ROWS, 128), jnp.float32)
    iota = jnp.arange(G_ROWS, dtype=jnp.int32)
    q = _sc_kernel(e1, s1, ids1, zeros, iota)
    return q[:N].reshape(N, 1)
